# baseline (speedup 1.0000x reference)
import jax
import jax.numpy as jnp
from jax.experimental import pallas as pl

_B, _T, _D1, _D2 = 16, 4096, 64, 4
_F = _D1 * _D2


def _row_kernel(x_ref, o_ref):
    x = x_ref[0]
    tt = jax.lax.broadcasted_iota(jnp.int32, (_T, 1), 0)
    m = jnp.isnan(x[:, 0:1])
    first = jnp.min(jnp.where(m, tt, _T))
    last = jnp.max(jnp.where(m, tt, -1))
    s = first - 1
    e = last + 1
    a = x_ref[0, pl.ds(s, 1), :]
    b = x_ref[0, pl.ds(e, 1), :]
    step = 1.0 / (e - s).astype(jnp.float32)
    w = (tt - s).astype(jnp.float32) * step
    in_gap = (tt > s) & (tt < e)
    vals = a + w * (b - a)
    o_ref[0] = jnp.where(in_gap, vals, x)


def kernel(x):
    xf = x.reshape(_B, _T, _F)
    out = pl.pallas_call(
        _row_kernel,
        grid=(_B,),
        in_specs=[pl.BlockSpec((1, _T, _F), lambda i: (i, 0, 0))],
        out_specs=pl.BlockSpec((1, _T, _F), lambda i: (i, 0, 0)),
        out_shape=jax.ShapeDtypeStruct((_B, _T, _F), jnp.float32),
    )(xf)
    return out.reshape(_B, _T, _D1, _D2)


# speedup vs baseline: 3.4861x; 3.4861x over previous
import jax
import jax.numpy as jnp
from jax.experimental import pallas as pl

_B, _T, _D1, _D2 = 16, 4096, 64, 4


def _row_kernel(y_ref, o_ref):
    y = y_ref[0]
    tt = jax.lax.broadcasted_iota(jnp.int32, (1, 1, _T), 2)
    m = jnp.isnan(y[0:1, 0:1, :])
    first = jnp.min(jnp.where(m, tt, _T))
    last = jnp.max(jnp.where(m, tt, -1))
    s = first - 1
    e = last + 1
    a = jnp.sum(jnp.where(tt == s, y, 0.0), axis=2, keepdims=True)
    b = jnp.sum(jnp.where(tt == e, y, 0.0), axis=2, keepdims=True)
    step = 1.0 / (e - s).astype(jnp.float32)
    w = (tt - s).astype(jnp.float32) * step
    in_gap = (tt > s) & (tt < e)
    vals = a + w * (b - a)
    o_ref[0] = jnp.where(in_gap, vals, y)


def kernel(x):
    y = jnp.transpose(x, (0, 2, 3, 1))
    out = pl.pallas_call(
        _row_kernel,
        grid=(_B,),
        in_specs=[pl.BlockSpec((1, _D1, _D2, _T), lambda i: (i, 0, 0, 0))],
        out_specs=pl.BlockSpec((1, _D1, _D2, _T), lambda i: (i, 0, 0, 0)),
        out_shape=jax.ShapeDtypeStruct((_B, _D1, _D2, _T), jnp.float32),
    )(y)
    return jnp.transpose(out, (0, 3, 1, 2))


# speedup vs baseline: 4.3166x; 1.2382x over previous
import jax
import jax.numpy as jnp
from jax.experimental import pallas as pl

_B, _T, _D1, _D2 = 16, 4096, 64, 4
_K = 2
_DB = _D1 // _K


def _row_kernel(det_ref, y_ref, o_ref):
    tt = jax.lax.broadcasted_iota(jnp.int32, (1, _T), 1)
    m = jnp.isnan(det_ref[0, 0, 0:1, :])
    first = jnp.min(jnp.where(m, tt, _T))
    last = jnp.max(jnp.where(m, tt, -1))
    s = first - 1
    e = last + 1
    base_s = pl.multiple_of((s // 128) * 128, 128)
    base_e = pl.multiple_of((e // 128) * 128, 128)
    y = y_ref[0]
    cs = y_ref[0, :, :, pl.ds(base_s, 128)]
    ce = y_ref[0, :, :, pl.ds(base_e, 128)]
    lane = jax.lax.broadcasted_iota(jnp.int32, (1, 1, 128), 2)
    a = jnp.sum(jnp.where(lane == s - base_s, cs, 0.0), axis=2, keepdims=True)
    b = jnp.sum(jnp.where(lane == e - base_e, ce, 0.0), axis=2, keepdims=True)
    tt3 = jax.lax.broadcasted_iota(jnp.int32, (1, 1, _T), 2)
    step = 1.0 / (e - s).astype(jnp.float32)
    w = (tt3 - s).astype(jnp.float32) * step
    in_gap = (tt3 > s) & (tt3 < e)
    vals = a + w * (b - a)
    o_ref[0] = jnp.where(in_gap, vals, y)


def kernel(x):
    y = jnp.transpose(x, (0, 2, 3, 1))
    out = pl.pallas_call(
        _row_kernel,
        grid=(_B, _K),
        in_specs=[
            pl.BlockSpec((1, 1, _D2, _T), lambda i, k: (i, 0, 0, 0)),
            pl.BlockSpec((1, _DB, _D2, _T), lambda i, k: (i, k, 0, 0)),
        ],
        out_specs=pl.BlockSpec((1, _DB, _D2, _T), lambda i, k: (i, k, 0, 0)),
        out_shape=jax.ShapeDtypeStruct((_B, _D1, _D2, _T), jnp.float32),
    )(y, y)
    return jnp.transpose(out, (0, 3, 1, 2))
